# baseline (speedup 1.0000x reference)
import functools

import jax
import jax.numpy as jnp
from jax import lax
from jax.experimental import pallas as pl
from jax.experimental.pallas import tpu as pltpu
from jax.experimental.pallas import tpu_sc as plsc

N = 10000
NP = 10240
F = 128
FH = 64
E = 320000
EP = 327680
C = 128
NCORE = 2
NSUB = 16
ROWS_PER_SUB = NP // NSUB
CHUNKS = EP // C
CHUNKS_PER_W = CHUNKS // (NCORE * NSUB)
CHUNKS_PER_T = CHUNKS // NSUB

_mesh = plsc.VectorSubcoreMesh(
    core_axis_name="c", subcore_axis_name="s",
    num_cores=NCORE, num_subcores=NSUB)


@functools.partial(
    pl.kernel,
    out_type=jax.ShapeDtypeStruct((NCORE, NP), jnp.float32),
    mesh=_mesh,
    scratch_types=[
        pltpu.VMEM_SHARED((NP,), jnp.float32),
        pltpu.VMEM((CHUNKS_PER_W, C), jnp.int32),
        pltpu.VMEM((C,), jnp.float32),
        pltpu.SemaphoreType.DMA,
    ],
)
def _deg_kernel(dstp, zeros1d, out, deg_sp, dst_v, ones_v, sem):
    c = lax.axis_index("c")
    s = lax.axis_index("s")
    for j in range(C // 16):
        ones_v[pl.ds(j * 16, 16)] = jnp.full((16,), 1.0, jnp.float32)
    base = (c * NSUB + s) * CHUNKS_PER_W
    pltpu.sync_copy(dstp.at[pl.ds(base, CHUNKS_PER_W)], dst_v)
    pltpu.sync_copy(zeros1d.at[pl.ds(s * ROWS_PER_SUB, ROWS_PER_SUB)],
                    deg_sp.at[pl.ds(s * ROWS_PER_SUB, ROWS_PER_SUB)])
    plsc.subcore_barrier()

    @pl.loop(0, CHUNKS_PER_W)
    def _(g):
        pltpu.async_copy(ones_v, deg_sp.at[dst_v.at[g]], sem, add=True)

    @pl.loop(0, CHUNKS_PER_W)
    def _(g):
        pltpu.make_async_copy(ones_v, deg_sp.at[dst_v.at[g]], sem).wait()

    plsc.subcore_barrier()
    pltpu.sync_copy(deg_sp.at[pl.ds(s * ROWS_PER_SUB, ROWS_PER_SUB)],
                    out.at[c, pl.ds(s * ROWS_PER_SUB, ROWS_PER_SUB)])


NGROUP = 4
GC = CHUNKS_PER_T // NGROUP

@functools.partial(
    pl.kernel,
    out_type=jax.ShapeDtypeStruct((NCORE, NP, FH), jnp.float32),
    mesh=_mesh,
    scratch_types=[
        pltpu.VMEM_SHARED((NP, FH), jnp.float32),
        pltpu.VMEM_SHARED((NP, FH), jnp.float32),
        pltpu.VMEM((2, GC, C), jnp.int32),
        pltpu.VMEM((2, GC, C), jnp.int32),
        pltpu.VMEM((3, C, FH), jnp.float32),
        [pltpu.SemaphoreType.DMA] * 2,
        [pltpu.SemaphoreType.DMA] * 3,
        [pltpu.SemaphoreType.DMA] * 3,
    ],
    compiler_params=pltpu.CompilerParams(use_tc_tiling_on_sc=False),
)
def _agg_kernel(hp, srcp, dstp, out, agg_sp, h_sp, src_v, dst_v,
                rows_v, isems, gsems, ssems):
    c = lax.axis_index("c")
    s = lax.axis_index("s")
    base = s * CHUNKS_PER_T
    rows = pl.ds(s * ROWS_PER_SUB, ROWS_PER_SUB)

    def start_group(gi, slot):
        pltpu.async_copy(srcp.at[pl.ds(base + gi * GC, GC)],
                         src_v.at[slot], isems[slot])
        pltpu.async_copy(dstp.at[pl.ds(base + gi * GC, GC)],
                         dst_v.at[slot], isems[slot])

    def wait_group(gi, slot):
        pltpu.make_async_copy(srcp.at[pl.ds(base, GC)], src_v.at[slot],
                              isems[slot]).wait()
        pltpu.make_async_copy(dstp.at[pl.ds(base, GC)], dst_v.at[slot],
                              isems[slot]).wait()

    start_group(0, 0)
    pltpu.sync_copy(hp.at[c].at[rows], h_sp.at[rows])
    pltpu.sync_copy(hp.at[c].at[rows], agg_sp.at[rows])
    plsc.subcore_barrier()

    def start_gather(slot, k, b):
        pltpu.async_copy(h_sp.at[src_v.at[slot, k]], rows_v.at[b], gsems[b])

    def wait_gather(slot, b):
        pltpu.make_async_copy(h_sp.at[src_v.at[slot, 0]], rows_v.at[b],
                              gsems[b]).wait()

    def start_scatter(slot, k, b):
        pltpu.async_copy(rows_v.at[b], agg_sp.at[dst_v.at[slot, k]],
                         ssems[b], add=True)

    def wait_scatter(slot, b):
        pltpu.make_async_copy(rows_v.at[b], agg_sp.at[dst_v.at[slot, 0]],
                              ssems[b]).wait()

    for gi in range(NGROUP):
        slot = gi % 2
        wait_group(gi, slot)
        if gi + 1 < NGROUP:
            start_group(gi + 1, (gi + 1) % 2)

        start_gather(slot, 0, 0)

        @pl.loop(0, GC)
        def _(k, slot=slot):
            b = lax.rem(k, 3)
            for bb in range(3):
                @pl.when(b == bb)
                def _(bb=bb):
                    bn = (bb + 1) % 3
                    @pl.when(k >= 2)
                    def _():
                        wait_scatter(slot, bn)

                    @pl.when(k + 1 < GC)
                    def _():
                        start_gather(slot, k + 1, bn)

                    wait_gather(slot, bb)
                    start_scatter(slot, k, bb)

        wait_scatter(slot, (GC - 2) % 3)
        wait_scatter(slot, (GC - 1) % 3)

    plsc.subcore_barrier()
    pltpu.sync_copy(agg_sp.at[rows], out.at[c, rows])


_RB = 2000


def _dinv_block(deg_ref):
    deg = deg_ref[0] + deg_ref[1] + 1.0
    return lax.rsqrt(deg)


def _tc_a_body(x_ref, w_ref, deg_ref, o_ref):
    dinv = _dinv_block(deg_ref)
    o_ref[0] = jnp.dot(x_ref[...], w_ref[0],
                       preferred_element_type=jnp.float32) * dinv


def _tc_b_body(agg_ref, deg_ref, w_ref, b_ref, o_ref):
    dinv = _dinv_block(deg_ref)
    full = jnp.concatenate([agg_ref[0], agg_ref[1]], axis=1)
    h1 = jnp.maximum(full * dinv + b_ref[...], 0.0)
    o_ref[0] = jnp.dot(h1, w_ref[0],
                       preferred_element_type=jnp.float32) * dinv


def _tc_c_body(agg_ref, deg_ref, b_ref, o_ref):
    dinv = _dinv_block(deg_ref)
    full = jnp.concatenate([agg_ref[0], agg_ref[1]], axis=1)
    o_ref[...] = jnp.maximum(full * dinv + b_ref[...], 0.0)


_row_spec = pl.BlockSpec((_RB, F), lambda i, j: (i, 0))
_half_spec = pl.BlockSpec((1, _RB, FH), lambda i, j: (j, i, 0))
_split_spec = pl.BlockSpec((NCORE, _RB, FH), lambda i, j: (0, i, 0))
_deg_spec = pl.BlockSpec((NCORE, _RB, 1), lambda i, j: (0, i, 0))
_whalf_spec = pl.BlockSpec((1, F, FH), lambda i, j: (j, 0, 0))
_b_spec = pl.BlockSpec((1, F), lambda i, j: (0, 0))
_SPLIT_OUT = jax.ShapeDtypeStruct((NCORE, NP, FH), jnp.float32)

_tc_a = pl.pallas_call(
    _tc_a_body, grid=(N // _RB, NCORE),
    in_specs=[_row_spec, _whalf_spec, _deg_spec],
    out_specs=_half_spec, out_shape=_SPLIT_OUT)

_tc_b = pl.pallas_call(
    _tc_b_body, grid=(N // _RB, NCORE),
    in_specs=[_split_spec, _deg_spec, _whalf_spec, _b_spec],
    out_specs=_half_spec, out_shape=_SPLIT_OUT)

_tc_c = pl.pallas_call(
    _tc_c_body, grid=(N // _RB, 1),
    in_specs=[_split_spec, _deg_spec, _b_spec],
    out_specs=pl.BlockSpec((_RB, F), lambda i, j: (i, 0)),
    out_shape=jax.ShapeDtypeStruct((N, F), jnp.float32))


def kernel(x, edge_index, W1, b1, W2, b2):
    src = edge_index[0]
    dst = edge_index[1]
    pad_idx = jnp.full((EP - E,), N, jnp.int32)
    srcp = jnp.concatenate([src, pad_idx]).reshape(CHUNKS, C)
    dstp = jnp.concatenate([dst, pad_idx]).reshape(CHUNKS, C)
    zeros1d = jnp.zeros((NP,), jnp.float32)
    b1r = b1.reshape(1, F)
    b2r = b2.reshape(1, F)
    W1s = W1.reshape(F, NCORE, FH).transpose(1, 0, 2)
    W2s = W2.reshape(F, NCORE, FH).transpose(1, 0, 2)

    degp = _deg_kernel(dstp, zeros1d).reshape(NCORE, NP, 1)
    h1p = _tc_a(x, W1s, degp)
    agg1 = _agg_kernel(h1p, srcp, dstp)
    h2p = _tc_b(agg1, degp, W2s, b1r)
    agg2 = _agg_kernel(h2p, srcp, dstp)
    return _tc_c(agg2, degp, b2r)
